# baseline (speedup 1.0000x reference)
import functools

import jax
import jax.numpy as jnp
from jax import lax
from jax.experimental import pallas as pl
from jax.experimental.pallas import tpu as pltpu
from jax.experimental.pallas import tpu_sc as plsc

_NC = 2
_NS = 16
_NW = _NC * _NS
_LANES = 128
_IR = 2
_CR = _IR * _LANES
_G = 20


def _pre_body(x_ref, wm_ref, bm_ref, wu2_ref, bu_ref, y_ref, z_ref):
    xb = x_ref[...]
    y_ref[...] = jnp.maximum(
        jnp.dot(xb, wm_ref[...], preferred_element_type=jnp.float32) + bm_ref[...],
        0.0)
    z_ref[...] = jnp.dot(xb, wu2_ref[...], preferred_element_type=jnp.float32) + bu_ref[...]


def _pre(x, Wm, bm2, Wu2, bu2):
    n, d = x.shape
    blk = 2000
    grid = n // blk
    return pl.pallas_call(
        _pre_body,
        grid=(grid,),
        in_specs=[
            pl.BlockSpec((blk, d), lambda i: (i, 0)),
            pl.BlockSpec((d, d), lambda i: (0, 0)),
            pl.BlockSpec((1, d), lambda i: (0, 0)),
            pl.BlockSpec((d, d), lambda i: (0, 0)),
            pl.BlockSpec((1, d), lambda i: (0, 0)),
        ],
        out_specs=[
            pl.BlockSpec((blk, d), lambda i: (i, 0)),
            pl.BlockSpec((blk, d), lambda i: (i, 0)),
        ],
        out_shape=[
            jax.ShapeDtypeStruct((n, d), jnp.float32),
            jax.ShapeDtypeStruct((n, d), jnp.float32),
        ],
    )(x, Wm, bm2, Wu2, bu2)


def _post_body(p0_ref, p1_ref, z_ref, wu1_ref, h_ref):
    agg = p0_ref[...] + p1_ref[...]
    h_ref[...] = jnp.maximum(
        jnp.dot(agg, wu1_ref[...], preferred_element_type=jnp.float32) + z_ref[...],
        0.0)


def _post(p0, p1, z, Wu1):
    n, d = z.shape
    blk = 2000
    grid = n // blk
    return pl.pallas_call(
        _post_body,
        grid=(grid,),
        in_specs=[
            pl.BlockSpec((blk, d), lambda i: (i, 0)),
            pl.BlockSpec((blk, d), lambda i: (i, 0)),
            pl.BlockSpec((blk, d), lambda i: (i, 0)),
            pl.BlockSpec((d, d), lambda i: (0, 0)),
        ],
        out_specs=pl.BlockSpec((blk, d), lambda i: (i, 0)),
        out_shape=jax.ShapeDtypeStruct((n, d), jnp.float32),
    )(p0, p1, z, Wu1)


def _sc_scatter(y, src_w, dst_w, zeros_pad, n, d, n_pad, chunks):
    rows_out = n_pad // _NS

    def body(y_hbm, s_hbm, d_hbm, zero_hbm, out_hbm, sbuf, dbuf, rows, agg_sh,
             *sems):
        c = lax.axis_index("c")
        s = lax.axis_index("s")
        wid = s * _NC + c

        pltpu.sync_copy(zero_hbm.at[pl.ds(s * rows_out, rows_out)],
                        agg_sh.at[pl.ds(s * rows_out, rows_out)])
        plsc.subcore_barrier()

        def group(g, carry):
            pltpu.sync_copy(s_hbm.at[wid, pl.ds(g * _G * _CR, _G * _CR)], sbuf)
            pltpu.sync_copy(d_hbm.at[wid, pl.ds(g * _G * _IR, _G * _IR)], dbuf)

            def step(jj, c2):
                pltpu.async_copy(y_hbm.at[sbuf.at[pl.ds(jj * _CR, _CR)]],
                                 rows, sems[0]).wait()
                for r in range(_IR):
                    pltpu.sync_copy(rows.at[pl.ds(r * _LANES, _LANES)],
                                    agg_sh.at[dbuf.at[jj * _IR + r]], add=True)
                return c2

            lax.fori_loop(0, _G, step, 0, unroll=False)
            return carry

        lax.fori_loop(0, chunks // _G, group, 0, unroll=False)
        plsc.subcore_barrier()

        pltpu.sync_copy(agg_sh.at[pl.ds(s * rows_out, rows_out)],
                        out_hbm.at[c, pl.ds(s * rows_out, rows_out)])

    mesh = plsc.VectorSubcoreMesh(core_axis_name="c", subcore_axis_name="s")
    f = pl.kernel(
        body,
        out_type=jax.ShapeDtypeStruct((_NC, n_pad, d), jnp.float32),
        mesh=mesh,
        scratch_types=[
            pltpu.VMEM((_G * _CR,), jnp.int32),
            pltpu.VMEM((_G * _IR, _LANES), jnp.int32),
            pltpu.VMEM((_CR, d), jnp.float32),
            pltpu.VMEM_SHARED((n_pad, d), jnp.float32),
            pltpu.SemaphoreType.DMA,
        ],
    )
    return f(y, src_w, dst_w, zeros_pad)


def kernel(x, edge_index, Wm, bm, Wu, bu):
    n, d = x.shape
    e = edge_index.shape[1]

    ept = -(-e // (_NW * _G * _CR)) * (_G * _CR)
    e_pad = ept * _NW
    chunks = ept // _CR
    n_pad = -(-(n + 1) // (_NS * 8)) * (_NS * 8)

    src = edge_index[0]
    dst = edge_index[1]
    pad = e_pad - e
    src_w = jnp.concatenate([src, jnp.zeros((pad,), jnp.int32)]).reshape(_NW, ept)
    dst_w = jnp.concatenate([dst, jnp.full((pad,), n, jnp.int32)]).reshape(
        _NW, chunks * _IR, _LANES)
    zeros_pad = jnp.zeros((n_pad, d), jnp.float32)

    bm2 = bm.reshape(1, d)
    bu2 = bu.reshape(1, d)
    Wu1 = Wu[:d]
    Wu2 = Wu[d:]

    y, z = _pre(x, Wm, bm2, Wu2, bu2)
    parts = _sc_scatter(y, src_w, dst_w, zeros_pad, n, d, n_pad, chunks)
    h = _post(parts[0, :n], parts[1, :n], z, Wu1)
    return h


# speedup vs baseline: 1.9481x; 1.9481x over previous
import functools

import jax
import jax.numpy as jnp
from jax import lax
from jax.experimental import pallas as pl
from jax.experimental.pallas import tpu as pltpu
from jax.experimental.pallas import tpu_sc as plsc

_NC = 2
_NS = 16
_NW = _NC * _NS
_LANES = 128
_F0 = 0.65


def _pre_body(x_ref, wm_ref, bm_ref, wu2_ref, bu_ref, y_ref, z_ref):
    xb = x_ref[...]
    y_ref[...] = jnp.maximum(
        jnp.dot(xb, wm_ref[...], preferred_element_type=jnp.float32) + bm_ref[...],
        0.0)
    z_ref[...] = jnp.dot(xb, wu2_ref[...], preferred_element_type=jnp.float32) + bu_ref[...]


def _pre(x, Wm, bm2, Wu2, bu2):
    n, d = x.shape
    blk = 2000
    grid = n // blk
    return pl.pallas_call(
        _pre_body,
        grid=(grid,),
        in_specs=[
            pl.BlockSpec((blk, d), lambda i: (i, 0)),
            pl.BlockSpec((d, d), lambda i: (0, 0)),
            pl.BlockSpec((1, d), lambda i: (0, 0)),
            pl.BlockSpec((d, d), lambda i: (0, 0)),
            pl.BlockSpec((1, d), lambda i: (0, 0)),
        ],
        out_specs=[
            pl.BlockSpec((blk, d), lambda i: (i, 0)),
            pl.BlockSpec((blk, d), lambda i: (i, 0)),
        ],
        out_shape=[
            jax.ShapeDtypeStruct((n, d), jnp.float32),
            jax.ShapeDtypeStruct((n, d), jnp.float32),
        ],
    )(x, Wm, bm2, Wu2, bu2)


def _post_body(p0_ref, p1_ref, z_ref, wu1_ref, h_ref):
    agg = p0_ref[...] + p1_ref[...]
    h_ref[...] = jnp.maximum(
        jnp.dot(agg, wu1_ref[...], preferred_element_type=jnp.float32) + z_ref[...],
        0.0)


def _post(p0, p1, z, Wu1):
    n, d = z.shape
    blk = 2000
    grid = n // blk
    return pl.pallas_call(
        _post_body,
        grid=(grid,),
        in_specs=[
            pl.BlockSpec((blk, d), lambda i: (i, 0)),
            pl.BlockSpec((blk, d), lambda i: (i, 0)),
            pl.BlockSpec((blk, d), lambda i: (i, 0)),
            pl.BlockSpec((d, d), lambda i: (0, 0)),
        ],
        out_specs=pl.BlockSpec((blk, d), lambda i: (i, 0)),
        out_shape=jax.ShapeDtypeStruct((n, d), jnp.float32),
    )(p0, p1, z, Wu1)


def _sc_scatter(y, src_w, dst_w, zeros_pad, n, d, n_pad, k0, k1):
    rows_out = n_pad // _NS
    kmax = max(k0, k1)

    def body(y_hbm, s_hbm, d_hbm, zero_hbm, out_hbm, idx_s, idx_d, rows,
             agg_sh, sem):
        c = lax.axis_index("c")
        s = lax.axis_index("s")
        wid = s * _NC + c

        pltpu.sync_copy(zero_hbm.at[pl.ds(s * rows_out, rows_out)],
                        agg_sh.at[pl.ds(s * rows_out, rows_out)])
        pltpu.sync_copy(s_hbm.at[wid], idx_s)
        pltpu.sync_copy(d_hbm.at[wid], idx_d)
        plsc.subcore_barrier()

        nch = jnp.where(c == 0, k0, k1)

        def step(j, carry):
            pltpu.async_copy(y_hbm.at[idx_s.at[j]], rows, sem).wait()
            pltpu.sync_copy(rows, agg_sh.at[idx_d.at[j]], add=True)
            return carry

        lax.fori_loop(0, nch, step, 0, unroll=False)
        plsc.subcore_barrier()

        pltpu.sync_copy(agg_sh.at[pl.ds(s * rows_out, rows_out)],
                        out_hbm.at[c, pl.ds(s * rows_out, rows_out)])

    mesh = plsc.VectorSubcoreMesh(core_axis_name="c", subcore_axis_name="s")
    f = pl.kernel(
        body,
        out_type=jax.ShapeDtypeStruct((_NC, n_pad, d), jnp.float32),
        mesh=mesh,
        scratch_types=[
            pltpu.VMEM((kmax, _LANES), jnp.int32),
            pltpu.VMEM((kmax, _LANES), jnp.int32),
            pltpu.VMEM((_LANES, d), jnp.float32),
            pltpu.VMEM_SHARED((n_pad, d), jnp.float32),
            pltpu.SemaphoreType.DMA,
        ],
    )
    return f(y, src_w, dst_w, zeros_pad)


def _pack_uneven(v, n_s, k0, k1, kmax, fill):
    c0 = v[:n_s * k0 * _LANES].reshape(n_s, k0, _LANES)
    c1 = v[n_s * k0 * _LANES:].reshape(n_s, k1, _LANES)
    if kmax > k0:
        c0 = jnp.concatenate(
            [c0, jnp.full((n_s, kmax - k0, _LANES), fill, jnp.int32)], axis=1)
    if kmax > k1:
        c1 = jnp.concatenate(
            [c1, jnp.full((n_s, kmax - k1, _LANES), fill, jnp.int32)], axis=1)
    return jnp.stack([c0, c1], axis=1).reshape(_NW, kmax, _LANES)


def kernel(x, edge_index, Wm, bm, Wu, bu):
    n, d = x.shape
    e = edge_index.shape[1]

    p = -(-e // (_NS * _LANES))
    e_pad = p * _NS * _LANES
    k0 = int(round(_F0 * p))
    k1 = p - k0
    kmax = max(k0, k1)
    n_pad = -(-(n + 1) // (_NS * 8)) * (_NS * 8)

    src = edge_index[0]
    dst = edge_index[1]
    pad = e_pad - e
    src_ext = jnp.concatenate([src, jnp.zeros((pad,), jnp.int32)])
    dst_ext = jnp.concatenate([dst, jnp.full((pad,), n, jnp.int32)])
    src_w = _pack_uneven(src_ext, _NS, k0, k1, kmax, 0)
    dst_w = _pack_uneven(dst_ext, _NS, k0, k1, kmax, n)
    zeros_pad = jnp.zeros((n_pad, d), jnp.float32)

    bm2 = bm.reshape(1, d)
    bu2 = bu.reshape(1, d)
    Wu1 = Wu[:d]
    Wu2 = Wu[d:]

    y, z = _pre(x, Wm, bm2, Wu2, bu2)
    parts = _sc_scatter(y, src_w, dst_w, zeros_pad, n, d, n_pad, k0, k1)
    h = _post(parts[0, :n], parts[1, :n], z, Wu1)
    return h
